# baseline (speedup 1.0000x reference)
import jax
import jax.numpy as jnp
from jax import lax
from jax.experimental import pallas as pl
from jax.experimental.pallas import tpu as pltpu
from jax.experimental.pallas import tpu_sc as plsc

N_NODES = 10000
N_EDGES = 160000
E_PAD = 163840
D_FEAT = 256
DQ = 64
NC = 2
NS = 16
EGP = E_PAD // 128
WG = EGP // (NC * NS)
HROWS = 10240
APT = HROWS // NS
HPT = N_NODES // NS
DUMMY = N_NODES
AROWS = N_NODES + 1
RING = 8
NCH = 10


def _hist_body(e_ref, out_ref, hist, didx, ones_v, zb, gsem):
    c = lax.axis_index("c")
    s = lax.axis_index("s")
    w = s * NC + c

    def fill_ones(i, _):
        ones_v[i] = jnp.ones((16,), jnp.float32)
        return 0

    lax.fori_loop(0, 128, fill_ones, 0)

    def fill_zero(i, _):
        zb[i] = jnp.zeros((16,), jnp.float32)
        return 0

    lax.fori_loop(0, APT, fill_zero, 0)

    pltpu.sync_copy(zb, hist.at[pl.ds(s * APT, APT)])
    plsc.subcore_barrier()

    pltpu.sync_copy(e_ref.at[1, pl.ds(w * WG, WG)], didx)
    cps = [
        pltpu.async_copy(ones_v, hist.at[didx.at[j]], gsem, add=True)
        for j in range(WG)
    ]
    for cp in cps:
        cp.wait()

    plsc.subcore_barrier()
    pltpu.sync_copy(hist.at[pl.ds(s * HPT, HPT)], zb.at[pl.ds(0, HPT)])
    pltpu.sync_copy(zb.at[pl.ds(0, HPT)], out_ref.at[c, pl.ds(s * HPT, HPT)])


def _sc_hist(er):
    mesh = plsc.VectorSubcoreMesh(core_axis_name="c", subcore_axis_name="s")
    return pl.kernel(
        _hist_body,
        out_type=jax.ShapeDtypeStruct((NC, N_NODES, 16), jnp.float32),
        mesh=mesh,
        scratch_types=[
            pltpu.VMEM_SHARED((HROWS, 16), jnp.float32),
            pltpu.VMEM((WG, 128), jnp.int32),
            pltpu.VMEM((128, 16), jnp.float32),
            pltpu.VMEM((APT, 16), jnp.float32),
            pltpu.SemaphoreType.DMA,
        ],
        compiler_params=pltpu.CompilerParams(use_tc_tiling_on_sc=False),
        name="gcn_sc_hist",
    )(er)


def _prescale_body(degp_ref, f1_ref, f2_ref, f3_ref, o1, o2, o3, on):
    deg = degp_ref[0, :, 0] + degp_ref[1, :, 0]
    nrm = lax.rsqrt(deg)[:, None]
    for f_ref, o in ((f1_ref, o1), (f2_ref, o2), (f3_ref, o3)):
        v = f_ref[...] * nrm
        for qq in range(4):
            o[qq] = v[:, qq * DQ:(qq + 1) * DQ]
    on[...] = nrm


def _tc_prescale(degp, f1, f2, f3):
    B = 1000
    grid = (N_NODES // B,)
    fspec = pl.BlockSpec((B, D_FEAT), lambda i: (i, 0))
    ospec = pl.BlockSpec((4, B, DQ), lambda i: (0, i, 0))
    oshape = jax.ShapeDtypeStruct((4, N_NODES, DQ), jnp.float32)
    return pl.pallas_call(
        _prescale_body,
        grid=grid,
        in_specs=[pl.BlockSpec((NC, B, 16), lambda i: (0, i, 0)),
                  fspec, fspec, fspec],
        out_specs=[ospec, ospec, ospec, pl.BlockSpec((B, 1), lambda i: (i, 0))],
        out_shape=[oshape, oshape, oshape,
                   jax.ShapeDtypeStruct((N_NODES, 1), jnp.float32)],
        name="gcn_tc_prescale",
    )(degp, f1, f2, f3)


def _agg_body(g1, g2, g3, e_ref, o1, o2, o3, acc, sidx, didx, rows,
              gsem, ssem, isem):
    c = lax.axis_index("c")
    s = lax.axis_index("s")

    def idx_load(ck):
        slot = lax.rem(ck, 4)
        base = 8 * s + 128 * ck
        pltpu.async_copy(
            e_ref.at[0, pl.ds(base, 8)], sidx.at[pl.ds(8 * slot, 8)], isem)
        pltpu.async_copy(
            e_ref.at[1, pl.ds(base, 8)], didx.at[pl.ds(8 * slot, 8)], isem)

    def idx_wait_bias(ck, coff):
        slot = lax.rem(ck, 4)
        base = 8 * s + 128 * ck
        pltpu.make_async_copy(
            e_ref.at[0, pl.ds(base, 8)], sidx.at[pl.ds(8 * slot, 8)], isem
        ).wait()
        pltpu.make_async_copy(
            e_ref.at[1, pl.ds(base, 8)], didx.at[pl.ds(8 * slot, 8)], isem
        ).wait()
        for r in range(8):
            for v in range(8):
                sl = (8 * slot + r, pl.ds(16 * v, 16))
                sidx[sl] = sidx[sl] + coff

    def g_copy(f_ref, ck, u, b):
        slot = lax.rem(ck, 4)
        return (f_ref.at[sidx.at[8 * slot + u]], rows.at[b], gsem)

    def s_copy(ck, u, b):
        slot = lax.rem(ck, 4)
        return (rows.at[b], acc.at[didx.at[8 * slot + u]], ssem)

    def fire_g(f_ref, ck, tb):
        for i in range(4):
            pltpu.async_copy(*g_copy(f_ref, ck, 4 * tb + i, 4 * tb + i))

    def drain_g(f_ref, ck, tb):
        for i in range(4):
            pltpu.make_async_copy(*g_copy(f_ref, ck, 4 * tb + i, 4 * tb + i)).wait()

    def fire_s(ck, tb):
        for i in range(4):
            pltpu.async_copy(*s_copy(ck, 4 * tb + i, 4 * tb + i), add=True)

    def drain_s(ck, tb):
        for i in range(4):
            pltpu.make_async_copy(*s_copy(ck, 4 * tb + i, 4 * tb + i)).wait()

    passes = []
    for f_ref, o_ref in ((g1, o1), (g2, o2), (g3, o3)):
        for q in range(2):
            passes.append((f_ref, o_ref, q))

    for f_ref, o_ref, q in passes:
        qq = c * 2 + q
        coff = qq * N_NODES

        def fill_zero(i, _):
            for v in range(4):
                rows[0, i, pl.ds(v * 16, 16)] = jnp.zeros((16,), jnp.float32)
            return 0

        lax.fori_loop(0, 128, fill_zero, 0)
        for z in range(4):
            pltpu.sync_copy(rows.at[0], acc.at[pl.ds(s * HPT + z * 128, 128)])
        pltpu.sync_copy(rows.at[0, pl.ds(0, HPT - 512)],
                        acc.at[pl.ds(s * HPT + 512, HPT - 512)])
        plsc.subcore_barrier()

        idx_load(0)
        idx_wait_bias(0, coff)
        idx_load(1)
        idx_wait_bias(1, coff)
        fire_g(f_ref, 0, 0)

        def kbody(kk, _):
            @pl.when(jnp.logical_and(kk >= 1, kk <= 8))
            def _():
                idx_wait_bias(kk + 1, coff)

            @pl.when(kk <= 7)
            def _():
                idx_load(kk + 2)

            drain_g(f_ref, kk, 0)

            @pl.when(kk > 0)
            def _():
                drain_s(kk - 1, 1)

            fire_s(kk, 0)
            fire_g(f_ref, kk, 1)

            drain_g(f_ref, kk, 1)
            drain_s(kk, 0)
            fire_s(kk, 1)

            @pl.when(kk <= 8)
            def _():
                fire_g(f_ref, kk + 1, 0)

            return 0

        lax.fori_loop(0, NCH, kbody, 0)
        drain_s(9, 1)

        plsc.subcore_barrier()
        stage = rows.at[0]
        for z in range(4):
            pltpu.sync_copy(acc.at[pl.ds(s * HPT + z * 128, 128)], stage)
            pltpu.sync_copy(stage, o_ref.at[qq, pl.ds(s * HPT + z * 128, 128)])
        st113 = rows.at[0, pl.ds(0, HPT - 512)]
        pltpu.sync_copy(acc.at[pl.ds(s * HPT + 512, HPT - 512)], st113)
        pltpu.sync_copy(st113, o_ref.at[qq, pl.ds(s * HPT + 512, HPT - 512)])


def _sc_agg(g1, g2, g3, er):
    mesh = plsc.VectorSubcoreMesh(core_axis_name="c", subcore_axis_name="s")
    out = jax.ShapeDtypeStruct((4, N_NODES, DQ), jnp.float32)
    return pl.kernel(
        _agg_body,
        out_type=(out, out, out),
        mesh=mesh,
        scratch_types=[
            pltpu.VMEM_SHARED((AROWS, DQ), jnp.float32),
            pltpu.VMEM((32, 128), jnp.int32),
            pltpu.VMEM((32, 128), jnp.int32),
            pltpu.VMEM((RING, 128, DQ), jnp.float32),
            pltpu.SemaphoreType.DMA,
            pltpu.SemaphoreType.DMA,
            pltpu.SemaphoreType.DMA,
        ],
        compiler_params=pltpu.CompilerParams(use_tc_tiling_on_sc=False),
        name="gcn_sc_agg",
    )(g1, g2, g3, er)


def _out_body(h1p, h2p, h3p, w1r, b1r, w2r, b2r, w3r, b3r, nr, o1, o2, o3):
    n2 = nr[...]
    for hp, wr, br, o in (
        (h1p, w1r, b1r, o1),
        (h2p, w2r, b2r, o2),
        (h3p, w3r, b3r, o3),
    ):
        acc = None
        for qq in range(4):
            d = lax.dot_general(
                hp[qq], wr[:, qq * DQ:(qq + 1) * DQ], (((1,), (1,)), ((), ())),
                preferred_element_type=jnp.float32,
            )
            acc = d if acc is None else acc + d
        o[...] = (acc + br[...][None, :]) * n2


def _tc_out(h1p, h2p, h3p, W1, b1, W2, b2, W3, b3, norm):
    B = 1000
    grid = (N_NODES // B,)
    hspec = pl.BlockSpec((4, B, DQ), lambda i: (0, i, 0))
    wspec = pl.BlockSpec((D_FEAT, D_FEAT), lambda i: (0, 0))
    bspec = pl.BlockSpec((D_FEAT,), lambda i: (0,))
    ospec = pl.BlockSpec((B, D_FEAT), lambda i: (i, 0))
    oshape = jax.ShapeDtypeStruct((N_NODES, D_FEAT), jnp.float32)
    return pl.pallas_call(
        _out_body,
        grid=grid,
        in_specs=[hspec, hspec, hspec, wspec, bspec, wspec, bspec, wspec, bspec,
                  pl.BlockSpec((B, 1), lambda i: (i, 0))],
        out_specs=[ospec, ospec, ospec],
        out_shape=[oshape, oshape, oshape],
        name="gcn_tc_out",
    )(h1p, h2p, h3p, W1, b1, W2, b2, W3, b3, norm)


@jax.jit
def kernel(feature1, feature2, feature3, edge_index, W1, b1, W2, b2, W3, b3):
    npad = E_PAD - N_EDGES
    pad = jnp.concatenate(
        [jnp.zeros((1, npad), jnp.int32),
         jnp.full((1, npad), DUMMY, jnp.int32)], axis=0)
    er = jnp.concatenate([edge_index, pad], axis=1).reshape(2, EGP, 128)
    degp = _sc_hist(er)
    fs1, fs2, fs3, norm = _tc_prescale(degp, feature1, feature2, feature3)
    h1p, h2p, h3p = _sc_agg(fs1.reshape(4 * N_NODES, DQ),
                            fs2.reshape(4 * N_NODES, DQ),
                            fs3.reshape(4 * N_NODES, DQ), er)
    return _tc_out(h1p, h2p, h3p, W1, b1, W2, b2, W3, b3, norm)


# speedup vs baseline: 1.8512x; 1.8512x over previous
import jax
import jax.numpy as jnp
from jax import lax
from jax.experimental import pallas as pl
from jax.experimental.pallas import tpu as pltpu
from jax.experimental.pallas import tpu_sc as plsc

N_NODES = 10000
N_EDGES = 160000
D_FEAT = 256
D_HALF = 128
NC = 2
NS = 16
EG = N_EDGES // 128
WG = 39
TG = 78
NPT = N_NODES // NS
DQ = 64


def _hist_body(e_ref, out_ref, hist, didx, ones_v, zb, gsem):
    c = lax.axis_index("c")
    s = lax.axis_index("s")
    w = s * NC + c

    def fill_ones(i, _):
        ones_v[i] = jnp.ones((16,), jnp.float32)
        return 0

    lax.fori_loop(0, 128, fill_ones, 0)

    def fill_zero(i, _):
        zb[i] = jnp.zeros((16,), jnp.float32)
        return 0

    lax.fori_loop(0, NPT, fill_zero, 0)

    pltpu.sync_copy(zb, hist.at[pl.ds(s * NPT, NPT)])
    plsc.subcore_barrier()

    pltpu.sync_copy(e_ref.at[1, pl.ds(w * WG, WG)], didx.at[pl.ds(0, WG)])
    cps = [
        pltpu.async_copy(ones_v, hist.at[didx.at[j]], gsem, add=True)
        for j in range(WG)
    ]
    for cp in cps:
        cp.wait()

    @pl.when(w < 2)
    def _():
        pltpu.sync_copy(e_ref.at[1, pl.ds(1248 + w, 1)], didx.at[pl.ds(WG, 1)])
        pltpu.async_copy(ones_v, hist.at[didx.at[WG]], gsem, add=True).wait()

    plsc.subcore_barrier()
    pltpu.sync_copy(hist.at[pl.ds(s * NPT, NPT)], zb)
    pltpu.sync_copy(zb, out_ref.at[c, pl.ds(s * NPT, NPT)])


def _sc_hist(er):
    mesh = plsc.VectorSubcoreMesh(core_axis_name="c", subcore_axis_name="s")
    return pl.kernel(
        _hist_body,
        out_type=jax.ShapeDtypeStruct((NC, N_NODES, 16), jnp.float32),
        mesh=mesh,
        scratch_types=[
            pltpu.VMEM_SHARED((N_NODES, 16), jnp.float32),
            pltpu.VMEM((WG + 1, 128), jnp.int32),
            pltpu.VMEM((128, 16), jnp.float32),
            pltpu.VMEM((NPT, 16), jnp.float32),
            pltpu.SemaphoreType.DMA,
        ],
        compiler_params=pltpu.CompilerParams(use_tc_tiling_on_sc=False),
        name="gcn_sc_hist",
    )(er)


def _prescale_body(degp_ref, f1_ref, f2_ref, f3_ref, o1, o2, o3, on):
    deg = degp_ref[0, :, 0] + degp_ref[1, :, 0]
    nrm = lax.rsqrt(deg)[:, None]
    o1[...] = f1_ref[...] * nrm
    o2[...] = f2_ref[...] * nrm
    o3[...] = f3_ref[...] * nrm
    on[...] = nrm


def _tc_prescale(degp, f1, f2, f3):
    B = 1000
    grid = (N_NODES // B,)
    return pl.pallas_call(
        _prescale_body,
        grid=grid,
        in_specs=[
            pl.BlockSpec((NC, B, 16), lambda i: (0, i, 0)),
            pl.BlockSpec((B, D_FEAT), lambda i: (i, 0)),
            pl.BlockSpec((B, D_FEAT), lambda i: (i, 0)),
            pl.BlockSpec((B, D_FEAT), lambda i: (i, 0)),
        ],
        out_specs=[
            pl.BlockSpec((B, D_FEAT), lambda i: (i, 0)),
            pl.BlockSpec((B, D_FEAT), lambda i: (i, 0)),
            pl.BlockSpec((B, D_FEAT), lambda i: (i, 0)),
            pl.BlockSpec((B, 1), lambda i: (i, 0)),
        ],
        out_shape=[
            jax.ShapeDtypeStruct((N_NODES, D_FEAT), jnp.float32),
            jax.ShapeDtypeStruct((N_NODES, D_FEAT), jnp.float32),
            jax.ShapeDtypeStruct((N_NODES, D_FEAT), jnp.float32),
            jax.ShapeDtypeStruct((N_NODES, 1), jnp.float32),
        ],
        name="gcn_tc_prescale",
    )(degp, f1, f2, f3)


def _agg_body(g1, g2, g3, e_ref, o1, o2, o3, acc, sraw, sidx, didx, rows,
              gsem, ssem):
    c = lax.axis_index("c")
    s = lax.axis_index("s")

    pltpu.sync_copy(e_ref.at[0, pl.ds(s * TG, TG)], sraw.at[pl.ds(0, TG)])
    pltpu.sync_copy(e_ref.at[1, pl.ds(s * TG, TG)], didx.at[pl.ds(0, TG)])

    @pl.when(s < 2)
    def _():
        pltpu.sync_copy(e_ref.at[0, pl.ds(1248 + s, 1)], sraw.at[pl.ds(TG, 1)])
        pltpu.sync_copy(e_ref.at[1, pl.ds(1248 + s, 1)], didx.at[pl.ds(TG, 1)])

    def fire_g(f_ref, t, half):
        for i in range(3):
            pltpu.async_copy(f_ref.at[sidx.at[t * 3 + i]], rows.at[half * 3 + i], gsem)

    def drain_g(f_ref, t, half):
        for i in range(3):
            pltpu.make_async_copy(
                f_ref.at[sidx.at[t * 3 + i]], rows.at[half * 3 + i], gsem
            ).wait()

    def fire_s(t, half):
        for i in range(3):
            pltpu.async_copy(
                rows.at[half * 3 + i], acc.at[didx.at[t * 3 + i]], ssem, add=True
            )

    def drain_s(t, half):
        for i in range(3):
            pltpu.make_async_copy(
                rows.at[half * 3 + i], acc.at[didx.at[t * 3 + i]], ssem
            ).wait()

    for f_ref, o_ref in ((g1, o1), (g2, o2), (g3, o3)):
        for q in range(2):
            qq = c * 2 + q

            def transform(j, _):
                for u in range(8):
                    sl = (j, pl.ds(u * 16, 16))
                    sidx[sl] = sraw[sl] * 4 + qq
                return 0

            lax.fori_loop(0, TG + 1, transform, 0)

            def fill_zero(i, _):
                for u in range(4):
                    rows[0, i, pl.ds(u * 16, 16)] = jnp.zeros((16,), jnp.float32)
                return 0

            lax.fori_loop(0, 128, fill_zero, 0)
            for z in range(4):
                pltpu.sync_copy(rows.at[0], acc.at[pl.ds(s * NPT + z * 128, 128)])
            pltpu.sync_copy(
                rows.at[0, pl.ds(0, NPT - 512)],
                acc.at[pl.ds(s * NPT + 512, NPT - 512)],
            )
            plsc.subcore_barrier()

            fire_g(f_ref, 0, 0)

            def kbody(kk, _):
                t0 = 2 * kk
                t1 = 2 * kk + 1
                drain_g(f_ref, t0, 0)

                @pl.when(t0 > 0)
                def _():
                    drain_s(t0 - 1, 1)

                fire_s(t0, 0)
                fire_g(f_ref, t1, 1)

                drain_g(f_ref, t1, 1)
                drain_s(t0, 0)
                fire_s(t1, 1)

                @pl.when(kk < 12)
                def _():
                    fire_g(f_ref, t1 + 1, 0)

                return 0

            lax.fori_loop(0, 13, kbody, 0)
            drain_s(25, 1)

            @pl.when(s < 2)
            def _():
                pltpu.async_copy(f_ref.at[sidx.at[TG]], rows.at[0], gsem).wait()
                pltpu.async_copy(
                    rows.at[0], acc.at[didx.at[TG]], ssem, add=True
                ).wait()

            plsc.subcore_barrier()
            for z in range(5):
                pltpu.sync_copy(
                    acc.at[pl.ds(s * NPT + z * 125, 125)], rows.at[0, pl.ds(0, 125)]
                )
                pltpu.sync_copy(
                    rows.at[0, pl.ds(0, 125)],
                    o_ref.at[qq, pl.ds(s * NPT + z * 125, 125)],
                )


def _sc_agg(g1, g2, g3, er):
    mesh = plsc.VectorSubcoreMesh(core_axis_name="c", subcore_axis_name="s")
    out = jax.ShapeDtypeStruct((4, N_NODES, DQ), jnp.float32)
    return pl.kernel(
        _agg_body,
        out_type=(out, out, out),
        mesh=mesh,
        scratch_types=[
            pltpu.VMEM_SHARED((N_NODES, DQ), jnp.float32),
            pltpu.VMEM((TG + 1, 128), jnp.int32),
            pltpu.VMEM((TG + 1, 128), jnp.int32),
            pltpu.VMEM((TG + 1, 128), jnp.int32),
            pltpu.VMEM((6, 128, DQ), jnp.float32),
            pltpu.SemaphoreType.DMA,
            pltpu.SemaphoreType.DMA,
        ],
        compiler_params=pltpu.CompilerParams(use_tc_tiling_on_sc=False),
        name="gcn_sc_agg",
    )(g1, g2, g3, er)


def _out_body(h1p, h2p, h3p, w1r, b1r, w2r, b2r, w3r, b3r, nr, o1, o2, o3):
    n2 = nr[...]
    for hp, wr, br, o in (
        (h1p, w1r, b1r, o1),
        (h2p, w2r, b2r, o2),
        (h3p, w3r, b3r, o3),
    ):
        acc = None
        for q in range(4):
            d = lax.dot_general(
                hp[q], wr[:, q * DQ:(q + 1) * DQ], (((1,), (1,)), ((), ())),
                preferred_element_type=jnp.float32,
            )
            acc = d if acc is None else acc + d
        o[...] = (acc + br[...][None, :]) * n2


def _tc_out(h1p, h2p, h3p, W1, b1, W2, b2, W3, b3, norm):
    B = 1000
    grid = (N_NODES // B,)
    hspec = pl.BlockSpec((4, B, DQ), lambda i: (0, i, 0))
    wspec = pl.BlockSpec((D_FEAT, D_FEAT), lambda i: (0, 0))
    bspec = pl.BlockSpec((D_FEAT,), lambda i: (0,))
    ospec = pl.BlockSpec((B, D_FEAT), lambda i: (i, 0))
    oshape = jax.ShapeDtypeStruct((N_NODES, D_FEAT), jnp.float32)
    return pl.pallas_call(
        _out_body,
        grid=grid,
        in_specs=[hspec, hspec, hspec, wspec, bspec, wspec, bspec, wspec, bspec,
                  pl.BlockSpec((B, 1), lambda i: (i, 0))],
        out_specs=[ospec, ospec, ospec],
        out_shape=[oshape, oshape, oshape],
        name="gcn_tc_out",
    )(h1p, h2p, h3p, W1, b1, W2, b2, W3, b3, norm)


@jax.jit
def kernel(feature1, feature2, feature3, edge_index, W1, b1, W2, b2, W3, b3):
    er = edge_index.reshape(2, EG, 128)
    degp = _sc_hist(er)
    fs1, fs2, fs3, norm = _tc_prescale(degp, feature1, feature2, feature3)
    g1 = fs1.reshape(4 * N_NODES, DQ)
    g2 = fs2.reshape(4 * N_NODES, DQ)
    g3 = fs3.reshape(4 * N_NODES, DQ)
    h1p, h2p, h3p = _sc_agg(g1, g2, g3, er)
    return _tc_out(h1p, h2p, h3p, W1, b1, W2, b2, W3, b3, norm)
